# baseline (speedup 1.0000x reference)
import functools

import jax
import jax.numpy as jnp
from jax import lax
from jax.experimental import pallas as pl
from jax.experimental.pallas import tpu as pltpu
from jax.experimental.pallas import tpu_sc as plsc

_N = 10000
_E = 160000
_D_IN = 256
_H = 512
_D_OUT = 256

_ROWS = 1000


def _p1_body(do_ref, di_ref, x_ref, ns_ref, nd_ref, g0_ref):
    ns = jax.lax.rsqrt(jnp.maximum(do_ref[...], 1.0))
    nd = jax.lax.rsqrt(jnp.maximum(di_ref[...], 1.0))
    ns_ref[...] = ns
    nd_ref[...] = nd
    g0_ref[...] = x_ref[...] * ns


def _p1(deg_out, deg_in, x):
    grid = (_N // _ROWS,)
    return pl.pallas_call(
        _p1_body,
        grid=grid,
        in_specs=[
            pl.BlockSpec((_ROWS, 1), lambda i: (i, 0)),
            pl.BlockSpec((_ROWS, 1), lambda i: (i, 0)),
            pl.BlockSpec((_ROWS, _D_IN), lambda i: (i, 0)),
        ],
        out_specs=[
            pl.BlockSpec((_ROWS, 1), lambda i: (i, 0)),
            pl.BlockSpec((_ROWS, 1), lambda i: (i, 0)),
            pl.BlockSpec((_ROWS, _D_IN), lambda i: (i, 0)),
        ],
        out_shape=[
            jax.ShapeDtypeStruct((_N, 1), jnp.float32),
            jax.ShapeDtypeStruct((_N, 1), jnp.float32),
            jax.ShapeDtypeStruct((_N, _D_IN), jnp.float32),
        ],
    )(deg_out, deg_in, x)


def _mm_body(a_ref, w_ref, b_ref, nd_ref, ns_ref, out_ref):
    t = jnp.dot(a_ref[...], w_ref[...], preferred_element_type=jnp.float32)
    h = jnp.maximum(t * nd_ref[...] + b_ref[...], 0.0)
    out_ref[...] = h * ns_ref[...]


def _p3(agg1, W0, b0, nd, ns):
    grid = (_N // _ROWS,)
    return pl.pallas_call(
        _mm_body,
        grid=grid,
        in_specs=[
            pl.BlockSpec((_ROWS, _D_IN), lambda i: (i, 0)),
            pl.BlockSpec((_D_IN, _H), lambda i: (0, 0)),
            pl.BlockSpec((1, _H), lambda i: (0, 0)),
            pl.BlockSpec((_ROWS, 1), lambda i: (i, 0)),
            pl.BlockSpec((_ROWS, 1), lambda i: (i, 0)),
        ],
        out_specs=pl.BlockSpec((_ROWS, _H), lambda i: (i, 0)),
        out_shape=jax.ShapeDtypeStruct((_N, _H), jnp.float32),
    )(agg1, W0, b0.reshape(1, _H), nd, ns)


def _p5_body(a_ref, w_ref, b_ref, nd_ref, wg_ref, bg_ref, out_ref, acc_ref):
    i = pl.program_id(0)
    t = jnp.dot(a_ref[...], w_ref[...], preferred_element_type=jnp.float32)
    h = jnp.maximum(t * nd_ref[...] + b_ref[...], 0.0)
    s = jnp.sum(h, axis=0, keepdims=True)

    @pl.when(i == 0)
    def _():
        acc_ref[...] = s

    @pl.when(i > 0)
    def _():
        acc_ref[...] = acc_ref[...] + s

    @pl.when(i == pl.num_programs(0) - 1)
    def _():
        out_ref[...] = (
            jnp.dot(acc_ref[...] * (1.0 / _N), wg_ref[...],
                    preferred_element_type=jnp.float32)
            + bg_ref[...]
        )


def _p5(agg2, W1, b1, nd, Wg, bg):
    grid = (_N // _ROWS,)
    return pl.pallas_call(
        _p5_body,
        grid=grid,
        in_specs=[
            pl.BlockSpec((_ROWS, _H), lambda i: (i, 0)),
            pl.BlockSpec((_H, _H), lambda i: (0, 0)),
            pl.BlockSpec((1, _H), lambda i: (0, 0)),
            pl.BlockSpec((_ROWS, 1), lambda i: (i, 0)),
            pl.BlockSpec((_H, _D_OUT), lambda i: (0, 0)),
            pl.BlockSpec((1, _D_OUT), lambda i: (0, 0)),
        ],
        out_specs=pl.BlockSpec((1, _D_OUT), lambda i: (0, 0)),
        out_shape=jax.ShapeDtypeStruct((1, _D_OUT), jnp.float32),
        scratch_shapes=[pltpu.VMEM((1, _H), jnp.float32)],
    )(agg2, W1, b1.reshape(1, _H), nd, Wg, bg.reshape(1, _D_OUT))


_NC = 2
_NS = 16
_L = 16
_EPT = _E // _NS
_B = 128
_NPAD = 10112
_HW = 16
_NAGG = 10240
_KMAX = 10240


def _sc_mesh():
    return plsc.VectorSubcoreMesh(core_axis_name="c", subcore_axis_name="s")


@functools.partial(
    pl.kernel,
    out_type=[
        jax.ShapeDtypeStruct((_NPAD, 128), jnp.float32),
        jax.ShapeDtypeStruct((_NPAD, 128), jnp.float32),
    ],
    mesh=_sc_mesh(),
    scratch_types=[
        pltpu.VMEM((_EPT + _B,), jnp.int32),
        pltpu.VMEM((_B,), jnp.int32),
        pltpu.VMEM((_B, 128), jnp.float32),
        pltpu.VMEM_SHARED((_NPAD, 128), jnp.float32),
    ],
)
def _p0_deg(src_hbm, dst_hbm, ones_hbm, zeros_hbm, dego_hbm, degi_hbm,
            idxf, idxw, ones_v, acc):
    c = lax.axis_index("c")
    s = lax.axis_index("s")
    rpt = _NPAD // _NS
    base_e = pl.multiple_of(s * _EPT, 8)
    pltpu.sync_copy(ones_hbm, ones_v)
    row0 = pl.multiple_of(s * rpt, 8)
    pltpu.sync_copy(zeros_hbm, acc.at[pl.ds(row0, rpt)])

    @pl.when(c == 0)
    def _():
        pltpu.sync_copy(src_hbm.at[pl.ds(base_e, _EPT)], idxf.at[pl.ds(0, _EPT)])

    @pl.when(c == 1)
    def _():
        pltpu.sync_copy(dst_hbm.at[pl.ds(base_e, _EPT)], idxf.at[pl.ds(0, _EPT)])

    pad = jnp.full((_L,), _N, jnp.int32)
    for k in range(_B // _L):
        idxf[pl.ds(_EPT + k * _L, _L)] = pad
    plsc.subcore_barrier()

    def _scat(j, carry):
        jb = pl.multiple_of(j * _B, _B)
        for k in range(_B // _L):
            idxw[pl.ds(k * _L, _L)] = idxf[pl.ds(jb + k * _L, _L)]
        pltpu.sync_copy(ones_v, acc.at[idxw], add=True)
        return carry

    nb = (_EPT + _B) // _B
    lax.fori_loop(0, nb, _scat, 0)
    plsc.subcore_barrier()

    @pl.when(c == 0)
    def _():
        pltpu.sync_copy(acc.at[pl.ds(row0, rpt)], dego_hbm.at[pl.ds(row0, rpt)])

    @pl.when(c == 1)
    def _():
        pltpu.sync_copy(acc.at[pl.ds(row0, rpt)], degi_hbm.at[pl.ds(row0, rpt)])


_EPT_P = 10240
_EPAD = _EPT_P * _NS
_FAR = 1 << 30
_W = 128
_BB = 128


_SEG = 2048
_BB = 64
_NBUF = 4


def _make_agg(nchunk):
    D = 256
    ns_strip = 2
    ch = _NAGG // nchunk
    acc_rows = ch + 128
    rounds = nchunk // _NC
    cpt = ch // _NS
    zpt = acc_rows // _NS
    nseg = _EPT_P // _SEG
    tps = (_SEG // _BB) * ns_strip
    KV = _BB // _L

    @functools.partial(
        pl.kernel,
        out_type=jax.ShapeDtypeStruct((_NAGG * ns_strip, _W), jnp.float32),
        mesh=_sc_mesh(),
        scratch_types=[
            pltpu.VMEM((_SEG,), jnp.int32),
            pltpu.VMEM((_SEG,), jnp.int32),
            pltpu.VMEM((_NBUF, _BB), jnp.int32),
            pltpu.VMEM((_NBUF, _BB), jnp.int32),
            pltpu.VMEM((_NBUF * _BB, _W), jnp.float32),
            pltpu.VMEM_SHARED((acc_rows * ns_strip, _W), jnp.float32),
            pltpu.SemaphoreType.DMA,
            pltpu.SemaphoreType.DMA,
            pltpu.SemaphoreType.DMA,
            pltpu.SemaphoreType.DMA,
            pltpu.SemaphoreType.DMA,
            pltpu.SemaphoreType.DMA,
            pltpu.SemaphoreType.DMA,
            pltpu.SemaphoreType.DMA,
        ],
    )
    def agg_kernel(gs_hbm, src_hbm, dst_hbm, zeros_hbm, out_hbm,
                   srcf, dstf, gidx, sidx, rows, acc,
                   sem0, sem1, sem2, sem3, sem4, sem5, sem6, sem7):
        sems = (sem0, sem1, sem2, sem3)
        ssems = (sem4, sem5, sem6, sem7)
        c = lax.axis_index("c")
        s = lax.axis_index("s")
        base_e = s * _EPT_P

        def _build_g(slot, jp, kkp):
            for k in range(KV):
                v = srcf[pl.ds(jp * _BB + k * _L, _L)]
                gidx[slot, pl.ds(k * _L, _L)] = v * ns_strip + kkp

        def _issue(slot):
            return pltpu.async_copy(
                gs_hbm.at[gidx.at[slot]],
                rows.at[pl.ds(slot * _BB, _BB)], sems[slot])

        def _consume(slot, j, kk, lo):
            pltpu.make_async_copy(
                gs_hbm.at[gidx.at[slot]],
                rows.at[pl.ds(slot * _BB, _BB)], sems[slot]).wait()
            for k in range(KV):
                d = dstf[pl.ds(j * _BB + k * _L, _L)]
                t = d - lo
                mi = 1 - lax.shift_right_logical(t | (ch - 1 - t), 31)
                tl = t * mi + (1 - mi) * ch
                sidx[slot, pl.ds(k * _L, _L)] = tl * ns_strip + kk
            pltpu.async_copy(rows.at[pl.ds(slot * _BB, _BB)],
                             acc.at[sidx.at[slot]], ssems[slot], add=True)

        def _wait_scatter(slot):
            pltpu.make_async_copy(rows.at[pl.ds(slot * _BB, _BB)],
                                  acc.at[sidx.at[slot]], ssems[slot]).wait()

        def _round(r, carry0):
            lo = (r * _NC + c) * ch
            zrow = pl.multiple_of(s * zpt * ns_strip, 8)
            pltpu.sync_copy(zeros_hbm, acc.at[pl.ds(zrow, zpt * ns_strip)])
            plsc.subcore_barrier()

            def _segment(si, carry1):
                soff = pl.multiple_of(base_e + si * _SEG, 8)
                pltpu.sync_copy(src_hbm.at[pl.ds(soff, _SEG)], srcf)
                pltpu.sync_copy(dst_hbm.at[pl.ds(soff, _SEG)], dstf)
                for slot in range(_NBUF):
                    _build_g(slot, slot // ns_strip, slot % ns_strip)
                for slot in range(_NBUF):
                    _issue(slot)

                def _body(m, carry2):
                    for slot in range(_NBUF):
                        j = (_NBUF // ns_strip) * m + slot // ns_strip
                        _consume(slot, j, slot % ns_strip, lo)
                    for slot in range(_NBUF):
                        jp = (_NBUF // ns_strip) * (m + 1) + slot // ns_strip
                        _wait_scatter(slot)
                        _build_g(slot, jp, slot % ns_strip)
                        _issue(slot)
                    return carry2

                nbody = (tps - _NBUF) // _NBUF
                lax.fori_loop(0, nbody, _body, 0)
                for slot in range(_NBUF):
                    j = (_NBUF // ns_strip) * nbody + slot // ns_strip
                    _consume(slot, j, slot % ns_strip, lo)
                for slot in range(_NBUF):
                    _wait_scatter(slot)
                return carry1

            lax.fori_loop(0, nseg, _segment, 0)
            plsc.subcore_barrier()
            out0 = pl.multiple_of(s * cpt * ns_strip, 8)
            pltpu.sync_copy(
                acc.at[pl.ds(out0, cpt * ns_strip)],
                out_hbm.at[pl.ds(lo * ns_strip + out0, cpt * ns_strip)])
            plsc.subcore_barrier()
            return carry0

        lax.fori_loop(0, rounds, _round, 0)

    return agg_kernel


_agg256 = _make_agg(2)


def _aggregate256(g, srcp, dstp):
    ns_strip = _D_IN // _W
    gs = g.reshape(_N * ns_strip, _W)
    zeros = jnp.zeros(((_NAGG // 2 + 128) // _NS * ns_strip, _W), jnp.float32)
    out = _agg256(gs, srcp, dstp, zeros)
    return out.reshape(_NAGG, _D_IN)[:_N]


def kernel(x, edge_index, W0, b0, W1, b1, Wg, bg):
    src = edge_index[0]
    dst = edge_index[1]
    ones16 = jnp.ones((_B, 128), jnp.float32)
    zeros16 = jnp.zeros((_NPAD // _NS, 128), jnp.float32)
    dego, degi = _p0_deg(src, dst, ones16, zeros16)
    deg_out = dego[:_N, 0:1]
    deg_in = degi[:_N, 0:1]
    ns, nd, g0 = _p1(deg_out, deg_in, x)
    srcp = jnp.concatenate([src, jnp.zeros((_EPAD - _E,), jnp.int32)])
    dstp = jnp.concatenate([dst, jnp.full((_EPAD - _E,), _FAR, jnp.int32)])
    agg1 = _aggregate256(g0, srcp, dstp)
    g1 = _p3(agg1, W0, b0, nd, ns)
    agg2 = jnp.concatenate(
        [_aggregate256(g1[:, :256], srcp, dstp),
         _aggregate256(g1[:, 256:], srcp, dstp)], axis=1)
    out = _p5(agg2, W1, b1, nd, Wg, bg)
    return out


# speedup vs baseline: 1.5239x; 1.5239x over previous
import functools

import jax
import jax.numpy as jnp
from jax import lax
from jax.experimental import pallas as pl
from jax.experimental.pallas import tpu as pltpu
from jax.experimental.pallas import tpu_sc as plsc

_N = 10000
_E = 160000
_D_IN = 256
_H = 512
_D_OUT = 256

_ROWS = 1000


def _p1_body(do_ref, di_ref, x_ref, ns_ref, nd_ref, g0_ref):
    ns = jax.lax.rsqrt(jnp.maximum(do_ref[...], 1.0))
    nd = jax.lax.rsqrt(jnp.maximum(di_ref[...], 1.0))
    ns_ref[...] = ns
    nd_ref[...] = nd
    g0_ref[...] = x_ref[...] * ns


def _p1(deg_out, deg_in, x):
    grid = (_N // _ROWS,)
    return pl.pallas_call(
        _p1_body,
        grid=grid,
        in_specs=[
            pl.BlockSpec((_ROWS, 1), lambda i: (i, 0)),
            pl.BlockSpec((_ROWS, 1), lambda i: (i, 0)),
            pl.BlockSpec((_ROWS, _D_IN), lambda i: (i, 0)),
        ],
        out_specs=[
            pl.BlockSpec((_ROWS, 1), lambda i: (i, 0)),
            pl.BlockSpec((_ROWS, 1), lambda i: (i, 0)),
            pl.BlockSpec((_ROWS, _D_IN), lambda i: (i, 0)),
        ],
        out_shape=[
            jax.ShapeDtypeStruct((_N, 1), jnp.float32),
            jax.ShapeDtypeStruct((_N, 1), jnp.float32),
            jax.ShapeDtypeStruct((_N, _D_IN), jnp.float32),
        ],
    )(deg_out, deg_in, x)


def _mm_body(a_ref, w_ref, b_ref, nd_ref, ns_ref, out_ref):
    t = jnp.dot(a_ref[...], w_ref[...], preferred_element_type=jnp.float32)
    h = jnp.maximum(t * nd_ref[...] + b_ref[...], 0.0)
    out_ref[...] = h * ns_ref[...]


def _p3(agg1, W0, b0, nd, ns):
    grid = (_N // _ROWS,)
    return pl.pallas_call(
        _mm_body,
        grid=grid,
        in_specs=[
            pl.BlockSpec((_ROWS, _D_IN), lambda i: (i, 0)),
            pl.BlockSpec((_D_IN, _H), lambda i: (0, 0)),
            pl.BlockSpec((1, _H), lambda i: (0, 0)),
            pl.BlockSpec((_ROWS, 1), lambda i: (i, 0)),
            pl.BlockSpec((_ROWS, 1), lambda i: (i, 0)),
        ],
        out_specs=pl.BlockSpec((_ROWS, _H), lambda i: (i, 0)),
        out_shape=jax.ShapeDtypeStruct((_N, _H), jnp.float32),
    )(agg1, W0, b0.reshape(1, _H), nd, ns)


def _p5_body(a_ref, w_ref, b_ref, nd_ref, wg_ref, bg_ref, out_ref, acc_ref):
    i = pl.program_id(0)
    t = jnp.dot(a_ref[...], w_ref[...], preferred_element_type=jnp.float32)
    h = jnp.maximum(t * nd_ref[...] + b_ref[...], 0.0)
    s = jnp.sum(h, axis=0, keepdims=True)

    @pl.when(i == 0)
    def _():
        acc_ref[...] = s

    @pl.when(i > 0)
    def _():
        acc_ref[...] = acc_ref[...] + s

    @pl.when(i == pl.num_programs(0) - 1)
    def _():
        out_ref[...] = (
            jnp.dot(acc_ref[...] * (1.0 / _N), wg_ref[...],
                    preferred_element_type=jnp.float32)
            + bg_ref[...]
        )


def _p5(agg2, W1, b1, nd, Wg, bg):
    grid = (_N // _ROWS,)
    return pl.pallas_call(
        _p5_body,
        grid=grid,
        in_specs=[
            pl.BlockSpec((_ROWS, _H), lambda i: (i, 0)),
            pl.BlockSpec((_H, _H), lambda i: (0, 0)),
            pl.BlockSpec((1, _H), lambda i: (0, 0)),
            pl.BlockSpec((_ROWS, 1), lambda i: (i, 0)),
            pl.BlockSpec((_H, _D_OUT), lambda i: (0, 0)),
            pl.BlockSpec((1, _D_OUT), lambda i: (0, 0)),
        ],
        out_specs=pl.BlockSpec((1, _D_OUT), lambda i: (0, 0)),
        out_shape=jax.ShapeDtypeStruct((1, _D_OUT), jnp.float32),
        scratch_shapes=[pltpu.VMEM((1, _H), jnp.float32)],
    )(agg2, W1, b1.reshape(1, _H), nd, Wg, bg.reshape(1, _D_OUT))


_NC = 2
_NS = 16
_L = 16
_EPT = _E // _NS
_B = 128
_NPAD = 10112
_HW = 16
_NAGG = 10240
_KMAX = 10240


def _sc_mesh():
    return plsc.VectorSubcoreMesh(core_axis_name="c", subcore_axis_name="s")


@functools.partial(
    pl.kernel,
    out_type=[
        jax.ShapeDtypeStruct((_NPAD, 128), jnp.float32),
        jax.ShapeDtypeStruct((_NPAD, 128), jnp.float32),
    ],
    mesh=_sc_mesh(),
    scratch_types=[
        pltpu.VMEM((_EPT + _B,), jnp.int32),
        pltpu.VMEM((_B,), jnp.int32),
        pltpu.VMEM((_B, 128), jnp.float32),
        pltpu.VMEM_SHARED((_NPAD, 128), jnp.float32),
    ],
)
def _p0_deg(src_hbm, dst_hbm, ones_hbm, zeros_hbm, dego_hbm, degi_hbm,
            idxf, idxw, ones_v, acc):
    c = lax.axis_index("c")
    s = lax.axis_index("s")
    rpt = _NPAD // _NS
    base_e = pl.multiple_of(s * _EPT, 8)
    pltpu.sync_copy(ones_hbm, ones_v)
    row0 = pl.multiple_of(s * rpt, 8)
    pltpu.sync_copy(zeros_hbm, acc.at[pl.ds(row0, rpt)])

    @pl.when(c == 0)
    def _():
        pltpu.sync_copy(src_hbm.at[pl.ds(base_e, _EPT)], idxf.at[pl.ds(0, _EPT)])

    @pl.when(c == 1)
    def _():
        pltpu.sync_copy(dst_hbm.at[pl.ds(base_e, _EPT)], idxf.at[pl.ds(0, _EPT)])

    pad = jnp.full((_L,), _N, jnp.int32)
    for k in range(_B // _L):
        idxf[pl.ds(_EPT + k * _L, _L)] = pad
    plsc.subcore_barrier()

    def _scat(j, carry):
        jb = pl.multiple_of(j * _B, _B)
        for k in range(_B // _L):
            idxw[pl.ds(k * _L, _L)] = idxf[pl.ds(jb + k * _L, _L)]
        pltpu.sync_copy(ones_v, acc.at[idxw], add=True)
        return carry

    nb = (_EPT + _B) // _B
    lax.fori_loop(0, nb, _scat, 0)
    plsc.subcore_barrier()

    @pl.when(c == 0)
    def _():
        pltpu.sync_copy(acc.at[pl.ds(row0, rpt)], dego_hbm.at[pl.ds(row0, rpt)])

    @pl.when(c == 1)
    def _():
        pltpu.sync_copy(acc.at[pl.ds(row0, rpt)], degi_hbm.at[pl.ds(row0, rpt)])


_EPT_P = 10240
_EPAD = _EPT_P * _NS
_FAR = 1 << 30
_W = 128
_BB = 128


_SEG = 2048
_BB = 64
_NBUF = 4
_EPT_S = _EPT_P
_ACC_R = _NAGG + 128


def _make_agg_fs(gstride, base):
    zpt = _ACC_R // _NS
    cpt = _NAGG // _NS
    tps = _SEG // _BB
    KV = _BB // _L
    nseg = _EPT_S // _SEG

    @functools.partial(
        pl.kernel,
        out_type=jax.ShapeDtypeStruct((_NC, _NAGG, _W), jnp.float32),
        mesh=_sc_mesh(),
        scratch_types=[
            pltpu.VMEM((_SEG,), jnp.int32),
            pltpu.VMEM((_SEG,), jnp.int32),
            pltpu.VMEM((_NBUF, _BB), jnp.int32),
            pltpu.VMEM((_BB,), jnp.int32),
            pltpu.VMEM((_NBUF * _BB, _W), jnp.float32),
            pltpu.VMEM_SHARED((_ACC_R, _W), jnp.float32),
            pltpu.SemaphoreType.DMA,
            pltpu.SemaphoreType.DMA,
            pltpu.SemaphoreType.DMA,
            pltpu.SemaphoreType.DMA,
        ],
    )
    def agg_kernel(gs_hbm, src_hbm, dst_hbm, zeros_hbm, out_hbm,
                   srcf, dstf, gidx, sidx, rows, acc,
                   sem0, sem1, sem2, sem3):
        sems = (sem0, sem1, sem2, sem3)
        c = lax.axis_index("c")
        s = lax.axis_index("s")
        goff = base + c
        base_e = s * _EPT_S
        zrow = pl.multiple_of(s * zpt, 8)
        pltpu.sync_copy(zeros_hbm, acc.at[pl.ds(zrow, zpt)])
        plsc.subcore_barrier()

        def _build_g(slot, jp):
            for k in range(KV):
                v = srcf[pl.ds(jp * _BB + k * _L, _L)]
                gidx[slot, pl.ds(k * _L, _L)] = v * gstride + goff

        def _issue(slot):
            return pltpu.async_copy(
                gs_hbm.at[gidx.at[slot]],
                rows.at[pl.ds(slot * _BB, _BB)], sems[slot])

        def _consume(slot, j):
            pltpu.make_async_copy(
                gs_hbm.at[gidx.at[slot]],
                rows.at[pl.ds(slot * _BB, _BB)], sems[slot]).wait()
            for k in range(KV):
                sidx[pl.ds(k * _L, _L)] = dstf[pl.ds(j * _BB + k * _L, _L)]
            pltpu.sync_copy(rows.at[pl.ds(slot * _BB, _BB)],
                            acc.at[sidx], add=True)

        def _segment(si, carry1):
            soff = pl.multiple_of(base_e + si * _SEG, 8)
            pltpu.sync_copy(src_hbm.at[pl.ds(soff, _SEG)], srcf)
            pltpu.sync_copy(dst_hbm.at[pl.ds(soff, _SEG)], dstf)
            for slot in range(_NBUF):
                _build_g(slot, slot)
            for slot in range(_NBUF):
                _issue(slot)

            def _body(m, carry2):
                for slot in range(_NBUF):
                    j = _NBUF * m + slot
                    _consume(slot, j)
                    _build_g(slot, j + _NBUF)
                    _issue(slot)
                return carry2

            nbody = (tps - _NBUF) // _NBUF
            lax.fori_loop(0, nbody, _body, 0)
            for slot in range(_NBUF):
                j = _NBUF * nbody + slot
                _consume(slot, j)
            return carry1

        lax.fori_loop(0, nseg, _segment, 0)
        plsc.subcore_barrier()
        out0 = pl.multiple_of(s * cpt, 8)
        pltpu.sync_copy(acc.at[pl.ds(out0, cpt)],
                        out_hbm.at[c, pl.ds(out0, cpt)])

    return agg_kernel


_agg_s2 = _make_agg_fs(2, 0)
_agg_s4l = _make_agg_fs(4, 0)
_agg_s4r = _make_agg_fs(4, 2)


def _zeros_acc():
    return jnp.zeros((_ACC_R // _NS, _W), jnp.float32)


def kernel(x, edge_index, W0, b0, W1, b1, Wg, bg):
    src = edge_index[0]
    dst = edge_index[1]
    ones16 = jnp.ones((_B, 128), jnp.float32)
    zeros16 = jnp.zeros((_NPAD // _NS, 128), jnp.float32)
    dego, degi = _p0_deg(src, dst, ones16, zeros16)
    deg_out = dego[:_N, 0:1]
    deg_in = degi[:_N, 0:1]
    ns, nd, g0 = _p1(deg_out, deg_in, x)
    srcp = jnp.concatenate([src, jnp.zeros((_EPAD - _E,), jnp.int32)])
    dstp = jnp.concatenate([dst, jnp.full((_EPAD - _E,), _NAGG, jnp.int32)])
    z = _zeros_acc()
    o1 = _agg_s2(g0.reshape(_N * 2, _W), srcp, dstp, z)
    agg1 = jnp.concatenate([o1[0, :_N], o1[1, :_N]], axis=1)
    g1 = _p3(agg1, W0, b0, nd, ns)
    g1s = g1.reshape(_N * 4, _W)
    ol = _agg_s4l(g1s, srcp, dstp, z)
    orr = _agg_s4r(g1s, srcp, dstp, z)
    agg2 = jnp.concatenate(
        [ol[0, :_N], ol[1, :_N], orr[0, :_N], orr[1, :_N]], axis=1)
    out = _p5(agg2, W1, b1, nd, Wg, bg)
    return out


# speedup vs baseline: 1.6743x; 1.0987x over previous
import functools

import jax
import jax.numpy as jnp
from jax import lax
from jax.experimental import pallas as pl
from jax.experimental.pallas import tpu as pltpu
from jax.experimental.pallas import tpu_sc as plsc

_N = 10000
_E = 160000
_D_IN = 256
_H = 512
_D_OUT = 256

_ROWS = 1000


def _p1_body(do_ref, di_ref, x_ref, ns_ref, nd_ref, g0_ref):
    ns = jax.lax.rsqrt(jnp.maximum(do_ref[...], 1.0))
    nd = jax.lax.rsqrt(jnp.maximum(di_ref[...], 1.0))
    ns_ref[...] = ns
    nd_ref[...] = nd
    g = (x_ref[...] * ns).reshape(_ROWS, 2, 128)
    g0_ref[...] = jnp.transpose(g, (1, 0, 2))


def _p1(deg_out, deg_in, x):
    grid = (_N // _ROWS,)
    return pl.pallas_call(
        _p1_body,
        grid=grid,
        in_specs=[
            pl.BlockSpec((_ROWS, 1), lambda i: (i, 0)),
            pl.BlockSpec((_ROWS, 1), lambda i: (i, 0)),
            pl.BlockSpec((_ROWS, _D_IN), lambda i: (i, 0)),
        ],
        out_specs=[
            pl.BlockSpec((_ROWS, 1), lambda i: (i, 0)),
            pl.BlockSpec((_ROWS, 1), lambda i: (i, 0)),
            pl.BlockSpec((2, _ROWS, 128), lambda i: (0, i, 0)),
        ],
        out_shape=[
            jax.ShapeDtypeStruct((_N, 1), jnp.float32),
            jax.ShapeDtypeStruct((_N, 1), jnp.float32),
            jax.ShapeDtypeStruct((2, _N, 128), jnp.float32),
        ],
    )(deg_out, deg_in, x)


def _mm_body(a_ref, w_ref, b_ref, nd_ref, ns_ref, out_ref):
    a = jnp.concatenate([a_ref[0], a_ref[1]], axis=1)
    t = jnp.dot(a, w_ref[...], preferred_element_type=jnp.float32)
    h = jnp.maximum(t * nd_ref[...] + b_ref[...], 0.0)
    g = (h * ns_ref[...]).reshape(_ROWS, 4, 128)
    out_ref[...] = jnp.transpose(g, (1, 0, 2))


def _p3(o1, W0, b0, nd, ns):
    grid = (_N // _ROWS,)
    return pl.pallas_call(
        _mm_body,
        grid=grid,
        in_specs=[
            pl.BlockSpec((2, _ROWS, 128), lambda i: (0, i, 0)),
            pl.BlockSpec((_D_IN, _H), lambda i: (0, 0)),
            pl.BlockSpec((1, _H), lambda i: (0, 0)),
            pl.BlockSpec((_ROWS, 1), lambda i: (i, 0)),
            pl.BlockSpec((_ROWS, 1), lambda i: (i, 0)),
        ],
        out_specs=pl.BlockSpec((4, _ROWS, 128), lambda i: (0, i, 0)),
        out_shape=jax.ShapeDtypeStruct((4, _N, 128), jnp.float32),
    )(o1, W0, b0.reshape(1, _H), nd, ns)


def _p5_body(al_ref, ar_ref, w_ref, b_ref, nd_ref, wg_ref, bg_ref, out_ref, acc_ref):
    i = pl.program_id(0)
    a = jnp.concatenate([al_ref[0], al_ref[1], ar_ref[0], ar_ref[1]], axis=1)
    t = jnp.dot(a, w_ref[...], preferred_element_type=jnp.float32)
    h = jnp.maximum(t * nd_ref[...] + b_ref[...], 0.0)
    s = jnp.sum(h, axis=0, keepdims=True)

    @pl.when(i == 0)
    def _():
        acc_ref[...] = s

    @pl.when(i > 0)
    def _():
        acc_ref[...] = acc_ref[...] + s

    @pl.when(i == pl.num_programs(0) - 1)
    def _():
        out_ref[...] = (
            jnp.dot(acc_ref[...] * (1.0 / _N), wg_ref[...],
                    preferred_element_type=jnp.float32)
            + bg_ref[...]
        )


def _p5(ol, orr, W1, b1, nd, Wg, bg):
    grid = (_N // _ROWS,)
    return pl.pallas_call(
        _p5_body,
        grid=grid,
        in_specs=[
            pl.BlockSpec((2, _ROWS, 128), lambda i: (0, i, 0)),
            pl.BlockSpec((2, _ROWS, 128), lambda i: (0, i, 0)),
            pl.BlockSpec((_H, _H), lambda i: (0, 0)),
            pl.BlockSpec((1, _H), lambda i: (0, 0)),
            pl.BlockSpec((_ROWS, 1), lambda i: (i, 0)),
            pl.BlockSpec((_H, _D_OUT), lambda i: (0, 0)),
            pl.BlockSpec((1, _D_OUT), lambda i: (0, 0)),
        ],
        out_specs=pl.BlockSpec((1, _D_OUT), lambda i: (0, 0)),
        out_shape=jax.ShapeDtypeStruct((1, _D_OUT), jnp.float32),
        scratch_shapes=[pltpu.VMEM((1, _H), jnp.float32)],
    )(ol, orr, W1, b1.reshape(1, _H), nd, Wg, bg.reshape(1, _D_OUT))


_NC = 2
_NS = 16
_L = 16
_EPT = _E // _NS
_B = 128
_NPAD = 10112
_HW = 16
_NAGG = 10240
_KMAX = 10240


def _sc_mesh():
    return plsc.VectorSubcoreMesh(core_axis_name="c", subcore_axis_name="s")


@functools.partial(
    pl.kernel,
    out_type=[
        jax.ShapeDtypeStruct((_NPAD, 128), jnp.float32),
        jax.ShapeDtypeStruct((_NPAD, 128), jnp.float32),
    ],
    mesh=_sc_mesh(),
    scratch_types=[
        pltpu.VMEM((_EPT + _B,), jnp.int32),
        pltpu.VMEM((_B,), jnp.int32),
        pltpu.VMEM((_B, 128), jnp.float32),
        pltpu.VMEM_SHARED((_NPAD, 128), jnp.float32),
    ],
)
def _p0_deg(src_hbm, dst_hbm, ones_hbm, zeros_hbm, dego_hbm, degi_hbm,
            idxf, idxw, ones_v, acc):
    c = lax.axis_index("c")
    s = lax.axis_index("s")
    rpt = _NPAD // _NS
    base_e = pl.multiple_of(s * _EPT, 8)
    pltpu.sync_copy(ones_hbm, ones_v)
    row0 = pl.multiple_of(s * rpt, 8)
    pltpu.sync_copy(zeros_hbm, acc.at[pl.ds(row0, rpt)])

    @pl.when(c == 0)
    def _():
        pltpu.sync_copy(src_hbm.at[pl.ds(base_e, _EPT)], idxf.at[pl.ds(0, _EPT)])

    @pl.when(c == 1)
    def _():
        pltpu.sync_copy(dst_hbm.at[pl.ds(base_e, _EPT)], idxf.at[pl.ds(0, _EPT)])

    pad = jnp.full((_L,), _N, jnp.int32)
    for k in range(_B // _L):
        idxf[pl.ds(_EPT + k * _L, _L)] = pad
    plsc.subcore_barrier()

    def _scat(j, carry):
        jb = pl.multiple_of(j * _B, _B)
        for k in range(_B // _L):
            idxw[pl.ds(k * _L, _L)] = idxf[pl.ds(jb + k * _L, _L)]
        pltpu.sync_copy(ones_v, acc.at[idxw], add=True)
        return carry

    nb = (_EPT + _B) // _B
    lax.fori_loop(0, nb, _scat, 0)
    plsc.subcore_barrier()

    @pl.when(c == 0)
    def _():
        pltpu.sync_copy(acc.at[pl.ds(row0, rpt)], dego_hbm.at[pl.ds(row0, rpt)])

    @pl.when(c == 1)
    def _():
        pltpu.sync_copy(acc.at[pl.ds(row0, rpt)], degi_hbm.at[pl.ds(row0, rpt)])


_EPT_P = 10240
_EPAD = _EPT_P * _NS
_FAR = 1 << 30
_W = 128
_BB = 128


_SEG = 2048
_BB = 64
_NBUF = 4
_EPT_S = _EPT_P
_ACC_R = _NAGG + 128


def _make_agg_fs(gstride, base):
    zpt = _ACC_R // _NS
    cpt = _NAGG // _NS
    tps = _SEG // _BB
    KV = _BB // _L
    nseg = _EPT_S // _SEG

    @functools.partial(
        pl.kernel,
        out_type=jax.ShapeDtypeStruct((_NC, _NAGG, _W), jnp.float32),
        mesh=_sc_mesh(),
        scratch_types=[
            pltpu.VMEM((_SEG,), jnp.int32),
            pltpu.VMEM((_SEG,), jnp.int32),
            pltpu.VMEM((_NBUF, _BB), jnp.int32),
            pltpu.VMEM((_BB,), jnp.int32),
            pltpu.VMEM((_NBUF * _BB, _W), jnp.float32),
            pltpu.VMEM_SHARED((_ACC_R, _W), jnp.float32),
            pltpu.SemaphoreType.DMA,
            pltpu.SemaphoreType.DMA,
            pltpu.SemaphoreType.DMA,
            pltpu.SemaphoreType.DMA,
        ],
    )
    def agg_kernel(gs_hbm, src_hbm, dst_hbm, zeros_hbm, out_hbm,
                   srcf, dstf, gidx, sidx, rows, acc,
                   sem0, sem1, sem2, sem3):
        sems = (sem0, sem1, sem2, sem3)
        c = lax.axis_index("c")
        s = lax.axis_index("s")
        goff = base + c
        base_e = s * _EPT_S
        zrow = pl.multiple_of(s * zpt, 8)
        pltpu.sync_copy(zeros_hbm, acc.at[pl.ds(zrow, zpt)])
        plsc.subcore_barrier()

        def _build_g(slot, jp):
            for k in range(KV):
                v = srcf[pl.ds(jp * _BB + k * _L, _L)]
                gidx[slot, pl.ds(k * _L, _L)] = goff * _N + v

        def _issue(slot):
            return pltpu.async_copy(
                gs_hbm.at[gidx.at[slot]],
                rows.at[pl.ds(slot * _BB, _BB)], sems[slot])

        def _consume(slot, j):
            pltpu.make_async_copy(
                gs_hbm.at[gidx.at[slot]],
                rows.at[pl.ds(slot * _BB, _BB)], sems[slot]).wait()
            for k in range(KV):
                sidx[pl.ds(k * _L, _L)] = dstf[pl.ds(j * _BB + k * _L, _L)]
            pltpu.sync_copy(rows.at[pl.ds(slot * _BB, _BB)],
                            acc.at[sidx], add=True)

        def _segment(si, carry1):
            soff = pl.multiple_of(base_e + si * _SEG, 8)
            pltpu.sync_copy(src_hbm.at[pl.ds(soff, _SEG)], srcf)
            pltpu.sync_copy(dst_hbm.at[pl.ds(soff, _SEG)], dstf)
            for slot in range(_NBUF):
                _build_g(slot, slot)
            for slot in range(_NBUF):
                _issue(slot)

            def _body(m, carry2):
                for slot in range(_NBUF):
                    j = _NBUF * m + slot
                    _consume(slot, j)
                    _build_g(slot, j + _NBUF)
                    _issue(slot)
                return carry2

            nbody = (tps - _NBUF) // _NBUF
            lax.fori_loop(0, nbody, _body, 0)
            for slot in range(_NBUF):
                j = _NBUF * nbody + slot
                _consume(slot, j)
            return carry1

        lax.fori_loop(0, nseg, _segment, 0)
        plsc.subcore_barrier()
        out0 = pl.multiple_of(s * cpt, 8)
        pltpu.sync_copy(acc.at[pl.ds(out0, cpt)],
                        out_hbm.at[c, pl.ds(out0, cpt)])

    return agg_kernel


_agg_s2 = _make_agg_fs(2, 0)
_agg_s4l = _make_agg_fs(4, 0)
_agg_s4r = _make_agg_fs(4, 2)


def _zeros_acc():
    return jnp.zeros((_ACC_R // _NS, _W), jnp.float32)


def kernel(x, edge_index, W0, b0, W1, b1, Wg, bg):
    src = edge_index[0]
    dst = edge_index[1]
    ones16 = jnp.ones((_B, 128), jnp.float32)
    zeros16 = jnp.zeros((_NPAD // _NS, 128), jnp.float32)
    dego, degi = _p0_deg(src, dst, ones16, zeros16)
    deg_out = dego[:_N, 0:1]
    deg_in = degi[:_N, 0:1]
    ns, nd, g0 = _p1(deg_out, deg_in, x)
    srcp = jnp.concatenate([src, jnp.zeros((_EPAD - _E,), jnp.int32)])
    dstp = jnp.concatenate([dst, jnp.full((_EPAD - _E,), _NAGG, jnp.int32)])
    z = _zeros_acc()
    o1 = _agg_s2(g0.reshape(2 * _N, _W), srcp, dstp, z)
    g1 = _p3(o1, W0, b0, nd, ns)
    g1s = g1.reshape(4 * _N, _W)
    ol = _agg_s4l(g1s, srcp, dstp, z)
    orr = _agg_s4r(g1s, srcp, dstp, z)
    out = _p5(ol, orr, W1, b1, nd, Wg, bg)
    return out
